# baseline (speedup 1.0000x reference)
import jax
import jax.numpy as jnp
from jax.experimental import pallas as pl
from jax.experimental.pallas import tpu as pltpu

D_MODEL = 768
D_MEMORY = 64
NUM_HEADS = 12
HEAD_DIM = D_MODEL // NUM_HEADS

_RT = (((1,), (1,)), ((), ()))
_CAN = (((1,), (0,)), ((), ()))

N_CHUNK = 4
N_BUF = 8
PREFETCH_BATCHES = 2


def _vcopy(mem_ref, buf_ref, sem_ref, b, c, m_chunk):
    slot = jax.lax.rem(b * N_CHUNK + c, N_BUF)
    return pltpu.make_async_copy(
        mem_ref.at[b, pl.ds(c * m_chunk, m_chunk), :],
        buf_ref.at[slot],
        sem_ref.at[slot])


def _attn_kernel(q_ref, mk_ref, mv_ref, wq_ref, bq_ref, wk_ref, wv_ref,
                 bv_ref, wo_ref, bo_ref, wg1_ref, bg1_ref, wg2_ref, bg2_ref,
                 out_ref, vbuf, sems):
    f32 = jnp.float32
    bf16 = jnp.bfloat16
    b = pl.program_id(0)
    n_b = pl.num_programs(0)
    m_chunk = vbuf.shape[1]

    @pl.when(b == 0)
    def _():
        for bb in range(PREFETCH_BATCHES):
            for c in range(N_CHUNK):
                _vcopy(mv_ref, vbuf, sems, bb, c, m_chunk).start()

    q32 = q_ref[0]
    s_len = q32.shape[0]
    qb = q32.astype(bf16)
    mkb = mk_ref[0].astype(bf16)

    scale = HEAD_DIM ** -0.5
    qp = (jax.lax.dot_general(qb, wq_ref[...], _RT, preferred_element_type=f32)
          + bq_ref[...]) * scale
    qpb = qp.astype(bf16)

    a48 = jnp.concatenate([
        jax.lax.dot_general(
            qpb[:, h * HEAD_DIM:(h + 1) * HEAD_DIM],
            wk_ref[h * HEAD_DIM:(h + 1) * HEAD_DIM, :],
            _CAN, preferred_element_type=f32)
        for h in range(NUM_HEADS)], axis=0)
    a48b = a48.astype(bf16)

    scores = jax.lax.dot_general(a48b, mkb, _RT,
                                 preferred_element_type=f32)
    mx = jnp.max(scores, axis=-1, keepdims=True)
    e = jnp.exp(scores - mx)
    denom = jnp.sum(e, axis=-1, keepdims=True)
    eb = e.astype(bf16)

    acc = jnp.zeros((s_len * NUM_HEADS, D_MEMORY), f32)
    for c in range(N_CHUNK):
        _vcopy(mv_ref, vbuf, sems, b, c, m_chunk).wait()
        slot = jax.lax.rem(b * N_CHUNK + c, N_BUF)
        vc = vbuf[slot].astype(bf16)
        acc = acc + jax.lax.dot_general(
            eb[:, c * m_chunk:(c + 1) * m_chunk], vc, _CAN,
            preferred_element_type=f32)
    r = acc / denom
    rb = r.astype(bf16)

    @pl.when(b + PREFETCH_BATCHES < n_b)
    def _():
        for c in range(N_CHUNK):
            _vcopy(mv_ref, vbuf, sems, b + PREFETCH_BATCHES, c,
                   m_chunk).start()

    ret = jnp.concatenate([
        jax.lax.dot_general(
            rb[h * s_len:(h + 1) * s_len, :],
            wv_ref[h * HEAD_DIM:(h + 1) * HEAD_DIM, :],
            _RT, preferred_element_type=f32)
        for h in range(NUM_HEADS)], axis=1)
    ret = (ret + bv_ref[...]).astype(bf16)

    ro = (jax.lax.dot_general(ret, wo_ref[...], _RT,
                              preferred_element_type=f32) + bo_ref[...])

    h1 = (jax.lax.dot_general(qb, wg1_ref[:, :D_MODEL], _RT,
                              preferred_element_type=f32)
          + jax.lax.dot_general(ro.astype(bf16), wg1_ref[:, D_MODEL:], _RT,
                                preferred_element_type=f32)
          + bg1_ref[...])
    h1 = h1 * jax.nn.sigmoid(h1)
    g = jax.nn.sigmoid(jnp.sum(h1 * wg2_ref[...], axis=-1, keepdims=True)
                       + bg2_ref[...])
    out_ref[0] = q32 + g * ro


def kernel(query, memory_keys, memory_values, Wq, bq, Wk, bk, Wv, bv,
           Wo, bo, Wg1, bg1, Wg2, bg2):
    b_sz, s_len, _ = query.shape
    m_sz = memory_keys.shape[1]
    m_chunk = m_sz // N_CHUNK
    bf16 = jnp.bfloat16
    del bk

    out = pl.pallas_call(
        _attn_kernel,
        grid=(b_sz,),
        in_specs=[
            pl.BlockSpec((1, s_len, D_MODEL), lambda b: (b, 0, 0)),
            pl.BlockSpec((1, m_sz, D_MEMORY), lambda b: (b, 0, 0)),
            pl.BlockSpec(memory_space=pltpu.MemorySpace.HBM),
            pl.BlockSpec((D_MODEL, D_MODEL), lambda b: (0, 0)),
            pl.BlockSpec((1, D_MODEL), lambda b: (0, 0)),
            pl.BlockSpec((D_MODEL, D_MEMORY), lambda b: (0, 0)),
            pl.BlockSpec((D_MODEL, D_MEMORY), lambda b: (0, 0)),
            pl.BlockSpec((1, D_MODEL), lambda b: (0, 0)),
            pl.BlockSpec((D_MODEL, D_MODEL), lambda b: (0, 0)),
            pl.BlockSpec((1, D_MODEL), lambda b: (0, 0)),
            pl.BlockSpec((D_MODEL, 2 * D_MODEL), lambda b: (0, 0)),
            pl.BlockSpec((1, D_MODEL), lambda b: (0, 0)),
            pl.BlockSpec((1, D_MODEL), lambda b: (0, 0)),
            pl.BlockSpec((1, 1), lambda b: (0, 0)),
        ],
        out_specs=pl.BlockSpec((1, s_len, D_MODEL), lambda b: (b, 0, 0)),
        out_shape=jax.ShapeDtypeStruct((b_sz, s_len, D_MODEL), jnp.float32),
        scratch_shapes=[
            pltpu.VMEM((N_BUF, m_chunk, D_MEMORY), jnp.float32),
            pltpu.SemaphoreType.DMA((N_BUF,)),
        ],
    )(query, memory_keys, memory_values,
      Wq.astype(bf16), bq.reshape(1, D_MODEL),
      Wk.astype(bf16), Wv.astype(bf16), bv.reshape(1, D_MODEL),
      Wo.astype(bf16), bo.reshape(1, D_MODEL),
      Wg1.astype(bf16), bg1.reshape(1, D_MODEL),
      Wg2.reshape(1, D_MODEL), bg2.reshape(1, 1))
    return out
